# baseline (speedup 1.0000x reference)
import functools

import jax
import jax.numpy as jnp
from jax import lax
from jax.experimental import pallas as pl
from jax.experimental.pallas import tpu as pltpu
from jax.experimental.pallas import tpu_sc as plsc

_K = 512
_D = 128
_G = 4
_C = _D // _G
_BETA = 0.5
_BR = 1024

_NW = 32
_CH = 2048


def _vq_tc_kernel(z_ref, cb2_ref, idx_ref, sse_ref, esq_ref):
    step = pl.program_id(0)

    @pl.when(step == 0)
    def _():
        for g in range(_G):
            E2 = cb2_ref[g]
            esq_ref[g, :] = 0.25 * jnp.sum(E2 * E2, axis=1)

    zb = z_ref[...]
    sse = jnp.zeros((1, 1), jnp.float32)
    for g in range(_G):
        zi = zb[:, g * _C:(g + 1) * _C]
        E2 = cb2_ref[g]
        a = jnp.sum(zi * zi, axis=1, keepdims=True)
        esq = esq_ref[g, :][None, :]
        mm2 = jax.lax.dot_general(
            zi, E2, (((1,), (1,)), ((), ())),
            precision=jax.lax.Precision.DEFAULT,
            preferred_element_type=jnp.float32)
        d = (a + esq) - mm2
        dmin = jnp.min(d, axis=1, keepdims=True)
        iota = jax.lax.broadcasted_iota(jnp.int32, d.shape, 1)
        idx = jnp.min(jnp.where(d == dmin, iota + g * _K, _G * _K), axis=1,
                      keepdims=True)
        idx_ref[:, g:g + 1] = idx
        sse = sse + jnp.sum(dmin).reshape(1, 1)

    @pl.when(step == 0)
    def _():
        sse_ref[...] = jnp.zeros((1, 1), jnp.float32)

    sse_ref[...] += sse


def _tc_indices(zr, codebooks, n):
    grid = (n // _BR,)
    return pl.pallas_call(
        _vq_tc_kernel,
        grid=grid,
        in_specs=[
            pl.BlockSpec((_BR, _D), lambda i: (i, 0)),
            pl.BlockSpec((_G, _K, _C), lambda i: (0, 0, 0)),
        ],
        out_specs=[
            pl.BlockSpec((_BR, _G), lambda i: (i, 0)),
            pl.BlockSpec((1, 1), lambda i: (0, 0)),
        ],
        out_shape=[
            jax.ShapeDtypeStruct((n, _G), jnp.int32),
            jax.ShapeDtypeStruct((1, 1), jnp.float32),
        ],
        scratch_shapes=[pltpu.VMEM((_G, _K), jnp.float32)],
    )(zr, codebooks * 2.0)


def _make_sc_gather(nrows):
    b_per_w = nrows // _NW
    nch = b_per_w // _CH
    mesh = plsc.VectorSubcoreMesh(core_axis_name="c", subcore_axis_name="s")

    @functools.partial(
        pl.kernel, mesh=mesh,
        out_type=jax.ShapeDtypeStruct((nrows, _C), jnp.float32),
        compiler_params=pltpu.CompilerParams(use_tc_tiling_on_sc=False),
        scratch_types=[
            pltpu.VMEM((nch, _CH), jnp.int32),
            pltpu.VMEM((_CH, _C), jnp.float32),
            pltpu.SemaphoreType.DMA,
        ],
    )
    def gather(idx_hbm, table_hbm, out_hbm, idx_v, rows_v, sem):
        wid = lax.axis_index("s") * 2 + lax.axis_index("c")
        base = wid * b_per_w
        pltpu.sync_copy(idx_hbm.at[wid], idx_v)
        for j in range(nch):
            pltpu.async_copy(table_hbm.at[idx_v.at[j]], rows_v, sem).wait()
            pltpu.sync_copy(rows_v, out_hbm.at[pl.ds(base + j * _CH, _CH)])

    return gather


def kernel(z, codebooks):
    n = z.shape[0] * z.shape[1]
    zr = z.reshape(n, _D)
    table = codebooks.reshape(_G * _K, _C)
    half = n // 2
    parts, sses = [], []
    for p in range(2):
        zp = jax.lax.slice(zr, (p * half, 0), ((p + 1) * half, _D))
        gidx, sse = _tc_indices(zp, codebooks, half)
        nrows = half * _G
        idx3 = gidx.reshape(_NW, (nrows // _NW) // _CH, _CH)
        parts.append(_make_sc_gather(nrows)(idx3, table))
        sses.append(sse)
    out = jnp.concatenate(parts, axis=0)
    sse = sses[0] + sses[1]
    loss = (sse[0, 0] * ((1.0 + _BETA) / (n * _D))).astype(jnp.float32)
    return (out.reshape(n, _D), loss)


# speedup vs baseline: 1.4487x; 1.4487x over previous
import functools

import jax
import jax.numpy as jnp
from jax import lax
from jax.experimental import pallas as pl
from jax.experimental.pallas import tpu as pltpu
from jax.experimental.pallas import tpu_sc as plsc

_K = 512
_D = 128
_G = 4
_C = _D // _G
_BETA = 0.5
_BR = 512

_NW = 32
_CH = 2048


def _vq_tc_kernel(z_ref, cb2_ref, idx_ref, sse_ref, esq_ref):
    step = pl.program_id(0)

    @pl.when(step == 0)
    def _():
        for g in range(_G):
            E2 = cb2_ref[g]
            esq_ref[g, :] = 0.25 * jnp.sum(E2 * E2, axis=1)

    zb = z_ref[...]
    sse = jnp.zeros((1, 1), jnp.float32)
    for g in range(_G):
        zi = zb[:, g * _C:(g + 1) * _C]
        E2 = cb2_ref[g]
        a = jnp.sum(zi * zi, axis=1, keepdims=True)
        esq = esq_ref[g, :][None, :]
        mm2 = jax.lax.dot_general(
            zi, E2, (((1,), (1,)), ((), ())),
            precision=jax.lax.Precision.DEFAULT,
            preferred_element_type=jnp.float32)
        d = (a + esq) - mm2
        dmin = jnp.min(d, axis=1, keepdims=True)
        iota = jax.lax.broadcasted_iota(jnp.int32, d.shape, 1)
        idx = jnp.min(jnp.where(d == dmin, iota + g * _K, _G * _K), axis=1,
                      keepdims=True)
        idx_ref[:, g:g + 1] = idx
        sse = sse + jnp.sum(dmin).reshape(1, 1)

    @pl.when(step == 0)
    def _():
        sse_ref[...] = jnp.zeros((1, 1), jnp.float32)

    sse_ref[...] += sse


def _tc_indices(zr, codebooks, n):
    grid = (n // _BR,)
    return pl.pallas_call(
        _vq_tc_kernel,
        grid=grid,
        in_specs=[
            pl.BlockSpec((_BR, _D), lambda i: (i, 0)),
            pl.BlockSpec((_G, _K, _C), lambda i: (0, 0, 0)),
        ],
        out_specs=[
            pl.BlockSpec((_BR, _G), lambda i: (i, 0)),
            pl.BlockSpec((1, 1), lambda i: (0, 0)),
        ],
        out_shape=[
            jax.ShapeDtypeStruct((n, _G), jnp.int32),
            jax.ShapeDtypeStruct((1, 1), jnp.float32),
        ],
        scratch_shapes=[pltpu.VMEM((_G, _K), jnp.float32)],
    )(zr, codebooks * 2.0)


def _make_sc_gather(nrows):
    b_per_w = nrows // _NW
    nch = b_per_w // _CH
    mesh = plsc.VectorSubcoreMesh(core_axis_name="c", subcore_axis_name="s")

    @functools.partial(
        pl.kernel, mesh=mesh,
        out_type=jax.ShapeDtypeStruct((nrows, _C), jnp.float32),
        compiler_params=pltpu.CompilerParams(use_tc_tiling_on_sc=False),
        scratch_types=[
            pltpu.VMEM((nch, _CH), jnp.int32),
            pltpu.VMEM((_CH, _C), jnp.float32),
            pltpu.SemaphoreType.DMA,
        ],
    )
    def gather(idx_hbm, table_hbm, out_hbm, idx_v, rows_v, sem):
        wid = lax.axis_index("s") * 2 + lax.axis_index("c")
        base = wid * b_per_w
        pltpu.sync_copy(idx_hbm.at[wid], idx_v)
        for j in range(nch):
            pltpu.async_copy(table_hbm.at[idx_v.at[j]], rows_v, sem).wait()
            pltpu.sync_copy(rows_v, out_hbm.at[pl.ds(base + j * _CH, _CH)])

    return gather


def kernel(z, codebooks):
    n = z.shape[0] * z.shape[1]
    zr = z.reshape(n, _D)
    gidx, sse = _tc_indices(zr, codebooks, n)
    nrows = n * _G
    table = codebooks.reshape(_G * _K, _C)
    idx3 = gidx.reshape(_NW, (nrows // _NW) // _CH, _CH)
    out = _make_sc_gather(nrows)(idx3, table)
    loss = (sse[0, 0] * ((1.0 + _BETA) / (n * _D))).astype(jnp.float32)
    return (out.reshape(n, _D), loss)


# speedup vs baseline: 1.6197x; 1.1181x over previous
import functools

import jax
import jax.numpy as jnp
from jax import lax
from jax.experimental import pallas as pl
from jax.experimental.pallas import tpu as pltpu
from jax.experimental.pallas import tpu_sc as plsc

_K = 512
_D = 128
_G = 4
_C = _D // _G
_BETA = 0.5
_BR = 1024

_NW = 32
_CH = 2048


def _vq_tc_kernel(z_ref, cb2_ref, idx_ref, sse_ref, esq_ref):
    step = pl.program_id(0)

    @pl.when(step == 0)
    def _():
        for g in range(_G):
            E2 = cb2_ref[g]
            esq_ref[g, :] = 0.25 * jnp.sum(E2 * E2, axis=1)

    sse = jnp.zeros((1, 1), jnp.float32)
    for g in range(_G):
        zi = z_ref[:, g * _C:(g + 1) * _C]
        E2 = cb2_ref[g]
        a = jnp.sum(zi * zi, axis=1, keepdims=True)
        esq = esq_ref[g, :][None, :]
        mm2 = jax.lax.dot_general(
            zi, E2, (((1,), (1,)), ((), ())),
            precision=jax.lax.Precision.DEFAULT,
            preferred_element_type=jnp.float32)
        d = (a + esq) - mm2
        dmin = jnp.min(d, axis=1, keepdims=True)
        iota = jax.lax.broadcasted_iota(jnp.int32, d.shape, 1)
        idx = jnp.min(jnp.where(d == dmin, iota + g * _K, _G * _K), axis=1,
                      keepdims=True)
        idx_ref[:, g:g + 1] = idx
        sse = sse + jnp.sum(dmin).reshape(1, 1)

    @pl.when(step == 0)
    def _():
        sse_ref[...] = jnp.zeros((1, 1), jnp.float32)

    sse_ref[...] += sse


def _tc_indices(zr, codebooks, n):
    grid = (n // _BR,)
    return pl.pallas_call(
        _vq_tc_kernel,
        grid=grid,
        in_specs=[
            pl.BlockSpec((_BR, _D), lambda i: (i, 0)),
            pl.BlockSpec((_G, _K, _C), lambda i: (0, 0, 0)),
        ],
        out_specs=[
            pl.BlockSpec((_BR, _G), lambda i: (i, 0)),
            pl.BlockSpec((1, 1), lambda i: (0, 0)),
        ],
        out_shape=[
            jax.ShapeDtypeStruct((n, _G), jnp.int32),
            jax.ShapeDtypeStruct((1, 1), jnp.float32),
        ],
        scratch_shapes=[pltpu.VMEM((_G, _K), jnp.float32)],
    )(zr, codebooks * 2.0)


def _make_sc_gather(nrows):
    b_per_w = nrows // _NW
    nch = b_per_w // _CH
    mesh = plsc.VectorSubcoreMesh(core_axis_name="c", subcore_axis_name="s")

    @functools.partial(
        pl.kernel, mesh=mesh,
        out_type=jax.ShapeDtypeStruct((nrows, _C), jnp.float32),
        compiler_params=pltpu.CompilerParams(use_tc_tiling_on_sc=False),
        scratch_types=[
            pltpu.VMEM((nch, _CH), jnp.int32),
            pltpu.VMEM((_CH, _C), jnp.float32),
            pltpu.SemaphoreType.DMA,
        ],
    )
    def gather(idx_hbm, table_hbm, out_hbm, idx_v, rows_v, sem):
        wid = lax.axis_index("s") * 2 + lax.axis_index("c")
        base = wid * b_per_w
        pltpu.sync_copy(idx_hbm.at[wid], idx_v)
        for j in range(nch):
            pltpu.async_copy(table_hbm.at[idx_v.at[j]], rows_v, sem).wait()
            pltpu.sync_copy(rows_v, out_hbm.at[pl.ds(base + j * _CH, _CH)])

    return gather


def kernel(z, codebooks):
    n = z.shape[0] * z.shape[1]
    zr = z.reshape(n, _D)
    gidx, sse = _tc_indices(zr, codebooks, n)
    nrows = n * _G
    table = codebooks.reshape(_G * _K, _C)
    idx3 = gidx.reshape(_NW, (nrows // _NW) // _CH, _CH)
    out = _make_sc_gather(nrows)(idx3, table)
    loss = (sse[0, 0] * ((1.0 + _BETA) / (n * _D))).astype(jnp.float32)
    return (out.reshape(n, _D), loss)


# speedup vs baseline: 1.6565x; 1.0227x over previous
import functools

import jax
import jax.numpy as jnp
from jax import lax
from jax.experimental import pallas as pl
from jax.experimental.pallas import tpu as pltpu
from jax.experimental.pallas import tpu_sc as plsc

_K = 512
_D = 128
_G = 4
_C = _D // _G
_BETA = 0.5
_BR = 1024

_NW = 32
_CH = 2048


def _vq_tc_kernel(z_ref, cb2_ref, idx_ref, sse_ref, esq_ref):
    step = pl.program_id(0)

    @pl.when(step == 0)
    def _():
        for g in range(_G):
            E2 = cb2_ref[g]
            esq_ref[g, :] = 0.25 * jnp.sum(E2 * E2, axis=1)

    sse = jnp.zeros((1, 1), jnp.float32)
    for g in range(_G):
        zi = z_ref[:, g * _C:(g + 1) * _C]
        E2 = cb2_ref[g]
        a = jnp.sum(zi * zi, axis=1, keepdims=True)
        esq = esq_ref[g, :][None, :]
        mm2 = jax.lax.dot_general(
            zi, E2, (((1,), (1,)), ((), ())),
            precision=jax.lax.Precision.DEFAULT,
            preferred_element_type=jnp.float32)
        d = (a + esq) - mm2
        dmin = jnp.min(d, axis=1, keepdims=True)
        iota = jax.lax.broadcasted_iota(
            jnp.int32, d.shape, 1).astype(jnp.float32)
        idxf = jnp.min(jnp.where(d == dmin, iota, float(_G * _K)), axis=1,
                       keepdims=True)
        idx_ref[:, g:g + 1] = idxf.astype(jnp.int32) + g * _K
        sse = sse + jnp.sum(dmin).reshape(1, 1)

    @pl.when(step == 0)
    def _():
        sse_ref[...] = jnp.zeros((1, 1), jnp.float32)

    sse_ref[...] += sse


def _tc_indices(zr, codebooks, n):
    grid = (n // _BR,)
    return pl.pallas_call(
        _vq_tc_kernel,
        grid=grid,
        in_specs=[
            pl.BlockSpec((_BR, _D), lambda i: (i, 0)),
            pl.BlockSpec((_G, _K, _C), lambda i: (0, 0, 0)),
        ],
        out_specs=[
            pl.BlockSpec((_BR, _G), lambda i: (i, 0)),
            pl.BlockSpec((1, 1), lambda i: (0, 0)),
        ],
        out_shape=[
            jax.ShapeDtypeStruct((n, _G), jnp.int32),
            jax.ShapeDtypeStruct((1, 1), jnp.float32),
        ],
        scratch_shapes=[pltpu.VMEM((_G, _K), jnp.float32)],
    )(zr, codebooks * 2.0)


def _make_sc_gather(nrows):
    b_per_w = nrows // _NW
    nch = b_per_w // _CH
    mesh = plsc.VectorSubcoreMesh(core_axis_name="c", subcore_axis_name="s")

    @functools.partial(
        pl.kernel, mesh=mesh,
        out_type=jax.ShapeDtypeStruct((nrows, _C), jnp.float32),
        compiler_params=pltpu.CompilerParams(use_tc_tiling_on_sc=False),
        scratch_types=[
            pltpu.VMEM((nch, _CH), jnp.int32),
            pltpu.VMEM((_CH, _C), jnp.float32),
            pltpu.SemaphoreType.DMA,
        ],
    )
    def gather(idx_hbm, table_hbm, out_hbm, idx_v, rows_v, sem):
        wid = lax.axis_index("s") * 2 + lax.axis_index("c")
        base = wid * b_per_w
        pltpu.sync_copy(idx_hbm.at[wid], idx_v)
        for j in range(nch):
            pltpu.async_copy(table_hbm.at[idx_v.at[j]], rows_v, sem).wait()
            pltpu.sync_copy(rows_v, out_hbm.at[pl.ds(base + j * _CH, _CH)])

    return gather


def kernel(z, codebooks):
    n = z.shape[0] * z.shape[1]
    zr = z.reshape(n, _D)
    gidx, sse = _tc_indices(zr, codebooks, n)
    nrows = n * _G
    table = codebooks.reshape(_G * _K, _C)
    idx3 = gidx.reshape(_NW, (nrows // _NW) // _CH, _CH)
    out = _make_sc_gather(nrows)(idx3, table)
    loss = (sse[0, 0] * ((1.0 + _BETA) / (n * _D))).astype(jnp.float32)
    return (out.reshape(n, _D), loss)
